# baseline (speedup 1.0000x reference)
import functools

import jax
import jax.numpy as jnp
from jax.experimental import pallas as pl
from jax.experimental.pallas import tpu as pltpu

HID = 128
NL = 3


def _linear(x, p):
    return x @ p["W"] + p["b"]


def _layer_norm(x, eps=1e-5):
    mu = jnp.mean(x, axis=-1, keepdims=True)
    var = jnp.mean((x - mu) ** 2, axis=-1, keepdims=True)
    return (x - mu) / jnp.sqrt(var + eps)


def _bn_eval(x, eps=1e-5):
    return x / jnp.sqrt(1.0 + eps)


def _genconv(x_src, x_dst, edge_index, p, n_dst, norm):
    src = edge_index[0]
    dst = edge_index[1]
    msg = jnp.maximum(x_src[src], 0.0) + 1e-7
    mmax = jax.ops.segment_max(msg, dst, num_segments=n_dst)
    mmax = jnp.where(jnp.isfinite(mmax), mmax, 0.0)
    ex = jnp.exp(msg - mmax[dst])
    denom = jax.ops.segment_sum(ex, dst, num_segments=n_dst)
    alpha = ex / denom[dst]
    aggr = jax.ops.segment_sum(msg * alpha, dst, num_segments=n_dst)
    out = aggr + x_dst
    h = _linear(out, p["l1"])
    h = _layer_norm(h) if norm == "layer" else _bn_eval(h)
    h = jnp.maximum(h, 0.0)
    return _linear(h, p["l2"])


def _pred_body(x_ref, w1_ref, b1_ref, w2_ref, b2_ref, o_ref):
    h = jnp.dot(x_ref[...], w1_ref[...], preferred_element_type=jnp.float32)
    h = jnp.maximum(h + b1_ref[...], 0.0)
    o = jnp.dot(h, w2_ref[...], preferred_element_type=jnp.float32)
    o_ref[...] = o + b2_ref[...]


def _pred_head(x, p1, p2, dout):
    n = x.shape[1]
    blk = 2000
    grid = (NL, n // blk)
    return pl.pallas_call(
        _pred_body,
        grid=grid,
        in_specs=[
            pl.BlockSpec((1, blk, HID), lambda i, j: (i, j, 0)),
            pl.BlockSpec((HID, HID), lambda i, j: (0, 0)),
            pl.BlockSpec((HID,), lambda i, j: (0,)),
            pl.BlockSpec((HID, dout), lambda i, j: (0, 0)),
            pl.BlockSpec((dout,), lambda i, j: (0,)),
        ],
        out_specs=pl.BlockSpec((1, blk, dout), lambda i, j: (i, j, 0)),
        out_shape=jax.ShapeDtypeStruct((NL, n, dout), jnp.float32),
    )(x, p1["W"], p1["b"], p2["W"], p2["b"])


def kernel(params, x_cons, x_vals, x_obj, ei_cons_to_vals, ei_vals_to_cons,
           ei_vals_to_obj, ei_obj_to_vals, ei_cons_to_obj, ei_obj_to_cons):
    NV = x_vals.shape[0]
    NC = x_cons.shape[0]
    NO = x_obj.shape[0]
    ei = {"cv": ei_cons_to_vals, "vc": ei_vals_to_cons, "vo": ei_vals_to_obj,
          "ov": ei_obj_to_vals, "co": ei_cons_to_obj, "oc": ei_obj_to_cons}
    x = {
        "cons": jnp.maximum(_linear(x_cons, params["enc_cons"]), 0.0),
        "vals": jnp.maximum(_linear(x_vals, params["enc_vals"]), 0.0),
        "obj": jnp.maximum(_linear(x_obj, params["enc_obj"]), 0.0),
    }
    hid_cons = []
    hid_vals = []
    for _ in range(NL):
        h1 = x
        h2 = {
            "vals": 0.5 * (_genconv(h1["cons"], h1["vals"], ei["cv"], params["conv_cv"], NV, "batch")
                           + _genconv(h1["obj"], h1["vals"], ei["ov"], params["conv_ov"], NV, "batch")),
            "cons": 0.5 * (_genconv(h1["vals"], h1["cons"], ei["vc"], params["conv_vc"], NC, "batch")
                           + _genconv(h1["obj"], h1["cons"], ei["oc"], params["conv_oc"], NC, "batch")),
            "obj": 0.5 * (_genconv(h1["vals"], h1["obj"], ei["vo"], params["conv_vo"], NO, "layer")
                          + _genconv(h1["cons"], h1["obj"], ei["co"], params["conv_co"], NO, "layer")),
        }
        hid_cons.append(h2["cons"])
        hid_vals.append(h2["vals"])
        x = {k: 0.5 * (jnp.maximum(h2[k], 0.0) + h1[k]) for k in ["cons", "vals", "obj"]}
    vals = jnp.stack(hid_vals, axis=0)
    cons = jnp.stack(hid_cons, axis=0)
    vals = _pred_head(vals, params["pred_vals_1"], params["pred_vals_2"], 2)
    cons = _pred_head(cons, params["pred_cons_1"], params["pred_cons_2"], 1)
    vals_out = jnp.transpose(vals, (1, 0, 2))
    cons_out = jnp.squeeze(cons, axis=-1).T
    return vals_out, cons_out


# speedup vs baseline: 1.6311x; 1.6311x over previous
import functools

import numpy as np
import jax
import jax.numpy as jnp
from jax import lax
from jax.experimental import pallas as pl
from jax.experimental.pallas import tpu as pltpu
from jax.experimental.pallas import tpu_sc as plsc

HID = 128
NL = 3
N = 10000
NT = 10240
HALF = NT // 2
HALFN = 5376
HSTRIPE = HALFN // 16
ZROWS = 48
CHUNK = 128
F32 = jnp.float32
I32 = jnp.int32
BN_SCALE = float(1.0 / np.sqrt(1.0 + 1e-5))


def _ceil_to(x, m):
    return ((x + m - 1) // m) * m


def _enc_body(x_ref, w_ref, b_ref, o_ref, g_ref):
    i = pl.program_id(0)
    h = jnp.dot(x_ref[...], w_ref[...], preferred_element_type=F32) + b_ref[...]
    h = jnp.maximum(h, 0.0)
    o_ref[...] = h
    cm = jnp.max(h, axis=0, keepdims=True)

    @pl.when(i == 0)
    def _():
        g_ref[...] = jnp.zeros((8, HID), F32)

    g_ref[...] = jnp.maximum(g_ref[...], jnp.broadcast_to(cm, (8, HID)))


def _encode(x, p):
    blk = 2048
    nin = x.shape[1]
    return pl.pallas_call(
        _enc_body,
        grid=(NT // blk,),
        in_specs=[
            pl.BlockSpec((blk, nin), lambda i: (i, 0)),
            pl.BlockSpec((nin, HID), lambda i: (0, 0)),
            pl.BlockSpec((HID,), lambda i: (0,)),
        ],
        out_specs=[
            pl.BlockSpec((blk, HID), lambda i: (i, 0)),
            pl.BlockSpec((8, HID), lambda i: (0, 0)),
        ],
        out_shape=[
            jax.ShapeDtypeStruct((NT, HID), F32),
            jax.ShapeDtypeStruct((8, HID), F32),
        ],
    )(x, p["W"], p["b"])


def _prep_body(x_ref, g_ref, t_ref):
    m = jnp.maximum(x_ref[...], 0.0) + 1e-7
    G = jnp.maximum(g_ref[0:1, :], 0.0) + 1e-7
    P = jnp.exp(m - G)
    Q = m * P
    t_ref[0] = jnp.concatenate([P[:, :64], Q[:, :64]], axis=1)
    t_ref[1] = jnp.concatenate([P[:, 64:], Q[:, 64:]], axis=1)


def _prep_tables(x, gmax):
    blk = 2048
    return pl.pallas_call(
        _prep_body,
        grid=(NT // blk,),
        in_specs=[
            pl.BlockSpec((blk, HID), lambda i: (i, 0)),
            pl.BlockSpec((8, HID), lambda i: (0, 0)),
        ],
        out_specs=pl.BlockSpec((2, blk, HID), lambda i: (0, i, 0)),
        out_shape=jax.ShapeDtypeStruct((2, NT, HID), F32),
    )(x, gmax)


def _combine_body(a1_ref, a2_ref, x_ref,
                  w11, b11, w12, b12, w21, b21, w22, b22,
                  h2_ref, xn_ref, g_ref, *, norm):
    x = x_ref[...]

    def conv(a_ref, w1, b1, w2, b2):
        c0 = a_ref[0]
        c1 = a_ref[1]
        den = jnp.concatenate([c0[:, :64], c1[:, :64]], axis=1)
        num = jnp.concatenate([c0[:, 64:], c1[:, 64:]], axis=1)
        aggr = jnp.where(den > 0, num / den, 0.0)
        o = aggr + x
        h = jnp.dot(o, w1[...], preferred_element_type=F32) + b1[...]
        if norm == "layer":
            mu = jnp.mean(h, axis=1, keepdims=True)
            var = jnp.mean((h - mu) ** 2, axis=1, keepdims=True)
            h = (h - mu) / jnp.sqrt(var + 1e-5)
        else:
            h = h * BN_SCALE
        h = jnp.maximum(h, 0.0)
        return jnp.dot(h, w2[...], preferred_element_type=F32) + b2[...]

    g1 = conv(a1_ref, w11, b11, w12, b12)
    g2 = conv(a2_ref, w21, b21, w22, b22)
    h2 = 0.5 * (g1 + g2)
    h2_ref[...] = h2
    xn = 0.5 * (jnp.maximum(h2, 0.0) + x)
    xn_ref[...] = xn
    i = pl.program_id(0)

    @pl.when(i == 0)
    def _():
        g_ref[...] = jnp.zeros((8, HID), F32)

    g_ref[...] = jnp.maximum(
        g_ref[...], jnp.broadcast_to(jnp.max(xn, axis=0, keepdims=True), (8, HID)))


def _combine(acc1, acc2, x, p1, p2, norm):
    blk = 2048
    body = functools.partial(_combine_body, norm=norm)
    wspec = lambda shp: pl.BlockSpec(shp, lambda i: tuple(0 for _ in shp))
    return pl.pallas_call(
        body,
        grid=(NT // blk,),
        in_specs=[
            pl.BlockSpec((2, blk, HID), lambda i: (0, i, 0)),
            pl.BlockSpec((2, blk, HID), lambda i: (0, i, 0)),
            pl.BlockSpec((blk, HID), lambda i: (i, 0)),
            wspec((HID, 2 * HID)), wspec((2 * HID,)),
            wspec((2 * HID, HID)), wspec((HID,)),
            wspec((HID, 2 * HID)), wspec((2 * HID,)),
            wspec((2 * HID, HID)), wspec((HID,)),
        ],
        out_specs=[
            pl.BlockSpec((blk, HID), lambda i: (i, 0)),
            pl.BlockSpec((blk, HID), lambda i: (i, 0)),
            pl.BlockSpec((8, HID), lambda i: (0, 0)),
        ],
        out_shape=[
            jax.ShapeDtypeStruct((NT, HID), F32),
            jax.ShapeDtypeStruct((NT, HID), F32),
            jax.ShapeDtypeStruct((8, HID), F32),
        ],
    )(acc1, acc2, x,
      p1["l1"]["W"], p1["l1"]["b"], p1["l2"]["W"], p1["l2"]["b"],
      p2["l1"]["W"], p2["l1"]["b"], p2["l2"]["W"], p2["l2"]["b"])


def _pred_body(x_ref, w1_ref, b1_ref, w2_ref, b2_ref, o_ref):
    h = jnp.dot(x_ref[0], w1_ref[...], preferred_element_type=F32)
    h = jnp.maximum(h + b1_ref[...], 0.0)
    o = jnp.dot(h, w2_ref[...], preferred_element_type=F32)
    o_ref[0] = o + b2_ref[...]


def _pred_head(x, p1, p2, dout):
    blk = 2048
    return pl.pallas_call(
        _pred_body,
        grid=(NL, NT // blk),
        in_specs=[
            pl.BlockSpec((1, blk, HID), lambda i, j: (i, j, 0)),
            pl.BlockSpec((HID, HID), lambda i, j: (0, 0)),
            pl.BlockSpec((HID,), lambda i, j: (0,)),
            pl.BlockSpec((HID, dout), lambda i, j: (0, 0)),
            pl.BlockSpec((dout,), lambda i, j: (0,)),
        ],
        out_specs=pl.BlockSpec((1, blk, dout), lambda i, j: (i, j, 0)),
        out_shape=jax.ShapeDtypeStruct((NL, NT, dout), F32),
    )(x, p1["W"], p1["b"], p2["W"], p2["b"])


def _make_agg(rows_list, table_ids, max_rows):
    nrel = len(rows_list)

    def body(*refs):
        tables = refs[0:3]
        idx = refs[3:3 + 4 * nrel]
        outs = refs[3 + 4 * nrel:3 + 5 * nrel]
        src_v, dst_v, buf, zbuf, acc = refs[-5:]
        c = lax.axis_index("c")
        s = lax.axis_index("s")

        def zrow(i, _):
            for k in range(8):
                zbuf[i, pl.ds(k * 16, 16)] = jnp.zeros((16,), F32)
            return 0

        lax.fori_loop(0, ZROWS, zrow, 0)

        for r, rows in enumerate(rows_list):
            tbl = tables[table_ids[r]]
            s0, s1, d0, d1 = idx[4 * r:4 * r + 4]
            out = outs[r]

            @pl.when(c == 0)
            def _():
                pltpu.sync_copy(s0.at[pl.ds(s * rows, rows)],
                                src_v.at[pl.ds(0, rows)])

            @pl.when(c == 1)
            def _():
                pltpu.sync_copy(s1.at[pl.ds(s * rows, rows)],
                                src_v.at[pl.ds(0, rows)])

            for p, d in enumerate((d0, d1)):
                for z in range(HSTRIPE // ZROWS):
                    pltpu.sync_copy(
                        zbuf, acc.at[pl.ds(s * HSTRIPE + z * ZROWS, ZROWS)])
                pltpu.sync_copy(d.at[pl.ds(s * rows, rows)],
                                dst_v.at[pl.ds(0, rows)])
                plsc.subcore_barrier()

                def step(j, _):
                    pltpu.sync_copy(tbl.at[src_v.at[j]], buf)
                    pltpu.sync_copy(buf, acc.at[dst_v.at[j]], add=True)
                    return 0

                lax.fori_loop(0, rows, step, 0)
                plsc.subcore_barrier()

                off = (2 * c + p) * HALFN + s * HSTRIPE
                pltpu.sync_copy(acc.at[pl.ds(s * HSTRIPE, HSTRIPE)],
                                out.at[pl.ds(off, HSTRIPE)])

    mesh = plsc.VectorSubcoreMesh(core_axis_name="c", subcore_axis_name="s")
    return pl.kernel(
        body,
        out_type=[jax.ShapeDtypeStruct((4 * HALFN, HID), F32)
                  for _ in rows_list],
        mesh=mesh,
        scratch_types=[
            pltpu.VMEM((max_rows, CHUNK), I32),
            pltpu.VMEM((max_rows, CHUNK), I32),
            pltpu.VMEM((CHUNK, HID), F32),
            pltpu.VMEM((ZROWS, HID), F32),
            pltpu.VMEM_SHARED((HALFN, HID), F32),
        ],
    )


def _pad_rows(x, n):
    return jnp.concatenate(
        [x, jnp.zeros((n - x.shape[0], x.shape[1]), x.dtype)], axis=0)


def _prep_edges(ei):
    e = ei.shape[1]
    ep = _ceil_to(e, 16 * CHUNK * 8)
    npad = ep - e
    src = ei[0].astype(I32)
    dst = ei[1].astype(I32)
    srcp = jnp.concatenate([src, jnp.full((npad,), N, I32)])
    dstp = jnp.concatenate(
        [dst, N + (jnp.arange(npad, dtype=I32) % (NT - N))])
    r = ep // CHUNK
    dummy = HALF + (jnp.arange(ep, dtype=I32) % (HALFN - HALF))
    d0 = jnp.where(dstp < HALF, dstp, dummy)
    d1 = jnp.where(dstp >= HALF, dstp - HALF, dummy)
    return (srcp.reshape(r, CHUNK), (srcp + NT).reshape(r, CHUNK),
            d0.reshape(r, CHUNK), d1.reshape(r, CHUNK), r // 16)


def kernel(params, x_cons, x_vals, x_obj, ei_cons_to_vals, ei_vals_to_cons,
           ei_vals_to_obj, ei_obj_to_vals, ei_cons_to_obj, ei_obj_to_cons):
    rels = [
        ("cv", 0, ei_cons_to_vals),
        ("ov", 2, ei_obj_to_vals),
        ("vc", 1, ei_vals_to_cons),
        ("oc", 2, ei_obj_to_cons),
        ("vo", 1, ei_vals_to_obj),
        ("co", 0, ei_cons_to_obj),
    ]
    idx_arrays = []
    rows_list = []
    for _, _, ei in rels:
        *arrs, rows = _prep_edges(ei)
        idx_arrays += arrs
        rows_list.append(rows)
    table_ids = [t for _, t, _ in rels]
    agg = _make_agg(tuple(rows_list), tuple(table_ids), max(rows_list))

    x = {
        "cons": _encode(_pad_rows(x_cons, NT), params["enc_cons"]),
        "vals": _encode(_pad_rows(x_vals, NT), params["enc_vals"]),
        "obj": _encode(_pad_rows(x_obj, NT), params["enc_obj"]),
    }

    def layer(carry, _):
        xs = carry
        tabs = [_prep_tables(*xs[t]).reshape(2 * NT, HID)
                for t in ("cons", "vals", "obj")]
        accs = agg(*tabs, *idx_arrays)
        accs = [a.reshape(2, 2, HALFN, HID)[:, :, :HALF, :].reshape(2, NT, HID)
                for a in accs]
        h2v, xv, gv = _combine(accs[0], accs[1], xs["vals"][0],
                               params["conv_cv"], params["conv_ov"], "batch")
        h2c, xc, gc = _combine(accs[2], accs[3], xs["cons"][0],
                               params["conv_vc"], params["conv_oc"], "batch")
        h2o, xo, go = _combine(accs[4], accs[5], xs["obj"][0],
                               params["conv_vo"], params["conv_co"], "layer")
        new = {"cons": (xc, gc), "vals": (xv, gv), "obj": (xo, go)}
        return new, (h2c, h2v)

    _, (cons, vals) = lax.scan(layer, x, None, length=NL)
    vals = _pred_head(vals, params["pred_vals_1"], params["pred_vals_2"], 2)
    cons = _pred_head(cons, params["pred_cons_1"], params["pred_cons_2"], 1)
    vals_out = jnp.transpose(vals[:, :N, :], (1, 0, 2))
    cons_out = jnp.squeeze(cons[:, :N, :], axis=-1).T
    return vals_out, cons_out


# speedup vs baseline: 3.2610x; 1.9992x over previous
import functools

import numpy as np
import jax
import jax.numpy as jnp
from jax import lax
from jax.experimental import pallas as pl
from jax.experimental.pallas import tpu as pltpu
from jax.experimental.pallas import tpu_sc as plsc

HID = 128
NL = 3
N = 10000
NT = 10240
NSTRIPE = NT // 16
ZROWS = 64
CHUNK = 128
IBLK = 16
F32 = jnp.float32
I32 = jnp.int32
BN_SCALE = float(1.0 / np.sqrt(1.0 + 1e-5))


def _ceil_to(x, m):
    return ((x + m - 1) // m) * m


def _enc_body(x_ref, w_ref, b_ref, o_ref, g_ref):
    i = pl.program_id(0)
    h = jnp.dot(x_ref[...], w_ref[...], preferred_element_type=F32) + b_ref[...]
    h = jnp.maximum(h, 0.0)
    o_ref[...] = h
    cm = jnp.max(h, axis=0, keepdims=True)

    @pl.when(i == 0)
    def _():
        g_ref[...] = jnp.zeros((8, HID), F32)

    g_ref[...] = jnp.maximum(g_ref[...], jnp.broadcast_to(cm, (8, HID)))


def _encode(x, p):
    blk = 2048
    nin = x.shape[1]
    return pl.pallas_call(
        _enc_body,
        grid=(NT // blk,),
        in_specs=[
            pl.BlockSpec((blk, nin), lambda i: (i, 0)),
            pl.BlockSpec((nin, HID), lambda i: (0, 0)),
            pl.BlockSpec((HID,), lambda i: (0,)),
        ],
        out_specs=[
            pl.BlockSpec((blk, HID), lambda i: (i, 0)),
            pl.BlockSpec((8, HID), lambda i: (0, 0)),
        ],
        out_shape=[
            jax.ShapeDtypeStruct((NT, HID), F32),
            jax.ShapeDtypeStruct((8, HID), F32),
        ],
    )(x, p["W"], p["b"])


def _prep_body(x_ref, g_ref, t_ref):
    m = jnp.maximum(x_ref[...], 0.0) + 1e-7
    G = jnp.maximum(g_ref[0:1, :], 0.0) + 1e-7
    P = jnp.exp(m - G)
    Q = m * P
    t_ref[0] = jnp.concatenate([P[:, :64], Q[:, :64]], axis=1)
    t_ref[1] = jnp.concatenate([P[:, 64:], Q[:, 64:]], axis=1)


def _prep_tables(x, gmax):
    blk = 2048
    return pl.pallas_call(
        _prep_body,
        grid=(NT // blk,),
        in_specs=[
            pl.BlockSpec((blk, HID), lambda i: (i, 0)),
            pl.BlockSpec((8, HID), lambda i: (0, 0)),
        ],
        out_specs=pl.BlockSpec((2, blk, HID), lambda i: (0, i, 0)),
        out_shape=jax.ShapeDtypeStruct((2, NT, HID), F32),
    )(x, gmax)


def _combine_body(a1_ref, a2_ref, x_ref,
                  w11, b11, w12, b12, w21, b21, w22, b22,
                  h2_ref, xn_ref, g_ref, *, norm):
    x = x_ref[...]

    def conv(a_ref, w1, b1, w2, b2):
        c0 = a_ref[0]
        c1 = a_ref[1]
        den = jnp.concatenate([c0[:, :64], c1[:, :64]], axis=1)
        num = jnp.concatenate([c0[:, 64:], c1[:, 64:]], axis=1)
        aggr = jnp.where(den > 0, num / den, 0.0)
        o = aggr + x
        h = jnp.dot(o, w1[...], preferred_element_type=F32) + b1[...]
        if norm == "layer":
            mu = jnp.mean(h, axis=1, keepdims=True)
            var = jnp.mean((h - mu) ** 2, axis=1, keepdims=True)
            h = (h - mu) / jnp.sqrt(var + 1e-5)
        else:
            h = h * BN_SCALE
        h = jnp.maximum(h, 0.0)
        return jnp.dot(h, w2[...], preferred_element_type=F32) + b2[...]

    g1 = conv(a1_ref, w11, b11, w12, b12)
    g2 = conv(a2_ref, w21, b21, w22, b22)
    h2 = 0.5 * (g1 + g2)
    h2_ref[...] = h2
    xn = 0.5 * (jnp.maximum(h2, 0.0) + x)
    xn_ref[...] = xn
    i = pl.program_id(0)

    @pl.when(i == 0)
    def _():
        g_ref[...] = jnp.zeros((8, HID), F32)

    g_ref[...] = jnp.maximum(
        g_ref[...], jnp.broadcast_to(jnp.max(xn, axis=0, keepdims=True), (8, HID)))


def _combine(acc1, acc2, x, p1, p2, norm):
    blk = 2048
    body = functools.partial(_combine_body, norm=norm)
    wspec = lambda shp: pl.BlockSpec(shp, lambda i: tuple(0 for _ in shp))
    return pl.pallas_call(
        body,
        grid=(NT // blk,),
        in_specs=[
            pl.BlockSpec((2, blk, HID), lambda i: (0, i, 0)),
            pl.BlockSpec((2, blk, HID), lambda i: (0, i, 0)),
            pl.BlockSpec((blk, HID), lambda i: (i, 0)),
            wspec((HID, 2 * HID)), wspec((2 * HID,)),
            wspec((2 * HID, HID)), wspec((HID,)),
            wspec((HID, 2 * HID)), wspec((2 * HID,)),
            wspec((2 * HID, HID)), wspec((HID,)),
        ],
        out_specs=[
            pl.BlockSpec((blk, HID), lambda i: (i, 0)),
            pl.BlockSpec((blk, HID), lambda i: (i, 0)),
            pl.BlockSpec((8, HID), lambda i: (0, 0)),
        ],
        out_shape=[
            jax.ShapeDtypeStruct((NT, HID), F32),
            jax.ShapeDtypeStruct((NT, HID), F32),
            jax.ShapeDtypeStruct((8, HID), F32),
        ],
    )(acc1, acc2, x,
      p1["l1"]["W"], p1["l1"]["b"], p1["l2"]["W"], p1["l2"]["b"],
      p2["l1"]["W"], p2["l1"]["b"], p2["l2"]["W"], p2["l2"]["b"])


def _pred_body(x_ref, w1_ref, b1_ref, w2_ref, b2_ref, o_ref):
    h = jnp.dot(x_ref[0], w1_ref[...], preferred_element_type=F32)
    h = jnp.maximum(h + b1_ref[...], 0.0)
    o = jnp.dot(h, w2_ref[...], preferred_element_type=F32)
    o_ref[0] = o + b2_ref[...]


def _pred_head(x, p1, p2, dout):
    blk = 2048
    return pl.pallas_call(
        _pred_body,
        grid=(NL, NT // blk),
        in_specs=[
            pl.BlockSpec((1, blk, HID), lambda i, j: (i, j, 0)),
            pl.BlockSpec((HID, HID), lambda i, j: (0, 0)),
            pl.BlockSpec((HID,), lambda i, j: (0,)),
            pl.BlockSpec((HID, dout), lambda i, j: (0, 0)),
            pl.BlockSpec((dout,), lambda i, j: (0,)),
        ],
        out_specs=pl.BlockSpec((1, blk, dout), lambda i, j: (i, j, 0)),
        out_shape=jax.ShapeDtypeStruct((NL, NT, dout), F32),
    )(x, p1["W"], p1["b"], p2["W"], p2["b"])


def _make_agg(rows_list, table_ids):
    nrel = len(rows_list)

    def body(*refs):
        tables = refs[0:3]
        idx = refs[3:3 + 3 * nrel]
        outs = refs[3 + 3 * nrel:3 + 4 * nrel]
        sblk, dblk, buf, zbuf, acc = refs[-5:]
        c = lax.axis_index("c")
        s = lax.axis_index("s")

        def zrow(i, _):
            for k in range(8):
                zbuf[i, pl.ds(k * 16, 16)] = jnp.zeros((16,), F32)
            return 0

        lax.fori_loop(0, ZROWS, zrow, 0)

        for r, rows in enumerate(rows_list):
            tbl = tables[table_ids[r]]
            s0, s1, d = idx[3 * r:3 * r + 3]
            out = outs[r]

            for z in range(NSTRIPE // ZROWS):
                pltpu.sync_copy(
                    zbuf, acc.at[pl.ds(s * NSTRIPE + z * ZROWS, ZROWS)])
            plsc.subcore_barrier()

            def block(b, _):
                base = s * rows + b * IBLK

                @pl.when(c == 0)
                def _():
                    pltpu.sync_copy(s0.at[pl.ds(base, IBLK)], sblk)

                @pl.when(c == 1)
                def _():
                    pltpu.sync_copy(s1.at[pl.ds(base, IBLK)], sblk)

                pltpu.sync_copy(d.at[pl.ds(base, IBLK)], dblk)
                for j in range(IBLK):
                    pltpu.sync_copy(tbl.at[sblk.at[j]], buf)
                    pltpu.sync_copy(buf, acc.at[dblk.at[j]], add=True)
                return 0

            lax.fori_loop(0, rows // IBLK, block, 0)
            plsc.subcore_barrier()

            pltpu.sync_copy(acc.at[pl.ds(s * NSTRIPE, NSTRIPE)],
                            out.at[pl.ds(c * NT + s * NSTRIPE, NSTRIPE)])

    mesh = plsc.VectorSubcoreMesh(core_axis_name="c", subcore_axis_name="s")
    return pl.kernel(
        body,
        out_type=[jax.ShapeDtypeStruct((2 * NT, HID), F32)
                  for _ in rows_list],
        mesh=mesh,
        scratch_types=[
            pltpu.VMEM((IBLK, CHUNK), I32),
            pltpu.VMEM((IBLK, CHUNK), I32),
            pltpu.VMEM((CHUNK, HID), F32),
            pltpu.VMEM((ZROWS, HID), F32),
            pltpu.VMEM_SHARED((NT, HID), F32),
        ],
    )


def _pad_rows(x, n):
    return jnp.concatenate(
        [x, jnp.zeros((n - x.shape[0], x.shape[1]), x.dtype)], axis=0)


def _prep_edges(ei):
    e = ei.shape[1]
    ep = _ceil_to(e, 16 * IBLK * CHUNK)
    npad = ep - e
    src = ei[0].astype(I32)
    dst = ei[1].astype(I32)
    srcp = jnp.concatenate([src, jnp.full((npad,), N, I32)])
    dstp = jnp.concatenate(
        [dst, N + (jnp.arange(npad, dtype=I32) % (NT - N))])
    r = ep // CHUNK
    return (srcp.reshape(r, CHUNK), (srcp + NT).reshape(r, CHUNK),
            dstp.reshape(r, CHUNK), r // 16)


def kernel(params, x_cons, x_vals, x_obj, ei_cons_to_vals, ei_vals_to_cons,
           ei_vals_to_obj, ei_obj_to_vals, ei_cons_to_obj, ei_obj_to_cons):
    rels = [
        ("cv", 0, ei_cons_to_vals),
        ("ov", 2, ei_obj_to_vals),
        ("vc", 1, ei_vals_to_cons),
        ("oc", 2, ei_obj_to_cons),
        ("vo", 1, ei_vals_to_obj),
        ("co", 0, ei_cons_to_obj),
    ]
    idx_arrays = []
    rows_list = []
    for _, _, ei in rels:
        *arrs, rows = _prep_edges(ei)
        idx_arrays += arrs
        rows_list.append(rows)
    table_ids = [t for _, t, _ in rels]
    agg = _make_agg(tuple(rows_list), tuple(table_ids))

    x = {
        "cons": _encode(_pad_rows(x_cons, NT), params["enc_cons"]),
        "vals": _encode(_pad_rows(x_vals, NT), params["enc_vals"]),
        "obj": _encode(_pad_rows(x_obj, NT), params["enc_obj"]),
    }

    def layer(carry, _):
        xs = carry
        tabs = [_prep_tables(*xs[t]).reshape(2 * NT, HID)
                for t in ("cons", "vals", "obj")]
        accs = agg(*tabs, *idx_arrays)
        accs = [a.reshape(2, NT, HID) for a in accs]
        h2v, xv, gv = _combine(accs[0], accs[1], xs["vals"][0],
                               params["conv_cv"], params["conv_ov"], "batch")
        h2c, xc, gc = _combine(accs[2], accs[3], xs["cons"][0],
                               params["conv_vc"], params["conv_oc"], "batch")
        h2o, xo, go = _combine(accs[4], accs[5], xs["obj"][0],
                               params["conv_vo"], params["conv_co"], "layer")
        new = {"cons": (xc, gc), "vals": (xv, gv), "obj": (xo, go)}
        return new, (h2c, h2v)

    _, (cons, vals) = lax.scan(layer, x, None, length=NL)
    vals = _pred_head(vals, params["pred_vals_1"], params["pred_vals_2"], 2)
    cons = _pred_head(cons, params["pred_cons_1"], params["pred_cons_2"], 1)
    vals_out = jnp.transpose(vals[:, :N, :], (1, 0, 2))
    cons_out = jnp.squeeze(cons[:, :N, :], axis=-1).T
    return vals_out, cons_out


# speedup vs baseline: 3.6690x; 1.1251x over previous
import functools

import numpy as np
import jax
import jax.numpy as jnp
from jax import lax
from jax.experimental import pallas as pl
from jax.experimental.pallas import tpu as pltpu
from jax.experimental.pallas import tpu_sc as plsc

HID = 128
NL = 3
N = 10000
NT = 10240
NSTRIPE = NT // 16
ZROWS = 64
CHUNK = 128
IBLK = 16
F32 = jnp.float32
I32 = jnp.int32
BN_SCALE = float(1.0 / np.sqrt(1.0 + 1e-5))


def _ceil_to(x, m):
    return ((x + m - 1) // m) * m


def _enc_body(x_ref, w_ref, b_ref, o_ref, g_ref):
    i = pl.program_id(0)
    h = jnp.dot(x_ref[...], w_ref[...], preferred_element_type=F32) + b_ref[...]
    h = jnp.maximum(h, 0.0)
    o_ref[...] = h
    cm = jnp.max(h, axis=0, keepdims=True)

    @pl.when(i == 0)
    def _():
        g_ref[...] = jnp.zeros((8, HID), F32)

    g_ref[...] = jnp.maximum(g_ref[...], jnp.broadcast_to(cm, (8, HID)))


def _encode(x, p):
    blk = 2048
    nin = x.shape[1]
    return pl.pallas_call(
        _enc_body,
        grid=(NT // blk,),
        in_specs=[
            pl.BlockSpec((blk, nin), lambda i: (i, 0)),
            pl.BlockSpec((nin, HID), lambda i: (0, 0)),
            pl.BlockSpec((HID,), lambda i: (0,)),
        ],
        out_specs=[
            pl.BlockSpec((blk, HID), lambda i: (i, 0)),
            pl.BlockSpec((8, HID), lambda i: (0, 0)),
        ],
        out_shape=[
            jax.ShapeDtypeStruct((NT, HID), F32),
            jax.ShapeDtypeStruct((8, HID), F32),
        ],
    )(x, p["W"], p["b"])


def _prep_body(x_ref, g_ref, t_ref):
    m = jnp.maximum(x_ref[...], 0.0) + 1e-7
    G = jnp.maximum(g_ref[0:1, :], 0.0) + 1e-7
    P = jnp.exp(m - G)
    Q = m * P
    t_ref[0] = jnp.concatenate([P[:, :64], Q[:, :64]], axis=1)
    t_ref[1] = jnp.concatenate([P[:, 64:], Q[:, 64:]], axis=1)


def _prep_tables(x, gmax):
    blk = 2048
    return pl.pallas_call(
        _prep_body,
        grid=(NT // blk,),
        in_specs=[
            pl.BlockSpec((blk, HID), lambda i: (i, 0)),
            pl.BlockSpec((8, HID), lambda i: (0, 0)),
        ],
        out_specs=pl.BlockSpec((2, blk, HID), lambda i: (0, i, 0)),
        out_shape=jax.ShapeDtypeStruct((2, NT, HID), F32),
    )(x, gmax)


def _combine_body(a1_ref, a2_ref, x_ref,
                  w11, b11, w12, b12, w21, b21, w22, b22,
                  h2_ref, xn_ref, g_ref, *, norm):
    x = x_ref[...]

    def conv(a_ref, w1, b1, w2, b2):
        c0 = a_ref[0]
        c1 = a_ref[1]
        den = jnp.concatenate([c0[:, :64], c1[:, :64]], axis=1)
        num = jnp.concatenate([c0[:, 64:], c1[:, 64:]], axis=1)
        aggr = jnp.where(den > 0, num / den, 0.0)
        o = aggr + x
        h = jnp.dot(o, w1[...], preferred_element_type=F32) + b1[...]
        if norm == "layer":
            mu = jnp.mean(h, axis=1, keepdims=True)
            var = jnp.mean((h - mu) ** 2, axis=1, keepdims=True)
            h = (h - mu) / jnp.sqrt(var + 1e-5)
        else:
            h = h * BN_SCALE
        h = jnp.maximum(h, 0.0)
        return jnp.dot(h, w2[...], preferred_element_type=F32) + b2[...]

    g1 = conv(a1_ref, w11, b11, w12, b12)
    g2 = conv(a2_ref, w21, b21, w22, b22)
    h2 = 0.5 * (g1 + g2)
    h2_ref[...] = h2
    xn = 0.5 * (jnp.maximum(h2, 0.0) + x)
    xn_ref[...] = xn
    i = pl.program_id(0)

    @pl.when(i == 0)
    def _():
        g_ref[...] = jnp.zeros((8, HID), F32)

    g_ref[...] = jnp.maximum(
        g_ref[...], jnp.broadcast_to(jnp.max(xn, axis=0, keepdims=True), (8, HID)))


def _combine(acc1, acc2, x, p1, p2, norm):
    blk = 2048
    body = functools.partial(_combine_body, norm=norm)
    wspec = lambda shp: pl.BlockSpec(shp, lambda i: tuple(0 for _ in shp))
    return pl.pallas_call(
        body,
        grid=(NT // blk,),
        in_specs=[
            pl.BlockSpec((2, blk, HID), lambda i: (0, i, 0)),
            pl.BlockSpec((2, blk, HID), lambda i: (0, i, 0)),
            pl.BlockSpec((blk, HID), lambda i: (i, 0)),
            wspec((HID, 2 * HID)), wspec((2 * HID,)),
            wspec((2 * HID, HID)), wspec((HID,)),
            wspec((HID, 2 * HID)), wspec((2 * HID,)),
            wspec((2 * HID, HID)), wspec((HID,)),
        ],
        out_specs=[
            pl.BlockSpec((blk, HID), lambda i: (i, 0)),
            pl.BlockSpec((blk, HID), lambda i: (i, 0)),
            pl.BlockSpec((8, HID), lambda i: (0, 0)),
        ],
        out_shape=[
            jax.ShapeDtypeStruct((NT, HID), F32),
            jax.ShapeDtypeStruct((NT, HID), F32),
            jax.ShapeDtypeStruct((8, HID), F32),
        ],
    )(acc1, acc2, x,
      p1["l1"]["W"], p1["l1"]["b"], p1["l2"]["W"], p1["l2"]["b"],
      p2["l1"]["W"], p2["l1"]["b"], p2["l2"]["W"], p2["l2"]["b"])


def _pred_body(x_ref, w1_ref, b1_ref, w2_ref, b2_ref, o_ref):
    h = jnp.dot(x_ref[0], w1_ref[...], preferred_element_type=F32)
    h = jnp.maximum(h + b1_ref[...], 0.0)
    o = jnp.dot(h, w2_ref[...], preferred_element_type=F32)
    o_ref[0] = o + b2_ref[...]


def _pred_head(x, p1, p2, dout):
    blk = 2048
    return pl.pallas_call(
        _pred_body,
        grid=(NL, NT // blk),
        in_specs=[
            pl.BlockSpec((1, blk, HID), lambda i, j: (i, j, 0)),
            pl.BlockSpec((HID, HID), lambda i, j: (0, 0)),
            pl.BlockSpec((HID,), lambda i, j: (0,)),
            pl.BlockSpec((HID, dout), lambda i, j: (0, 0)),
            pl.BlockSpec((dout,), lambda i, j: (0,)),
        ],
        out_specs=pl.BlockSpec((1, blk, dout), lambda i, j: (i, j, 0)),
        out_shape=jax.ShapeDtypeStruct((NL, NT, dout), F32),
    )(x, p1["W"], p1["b"], p2["W"], p2["b"])


def _make_agg(rows_list, table_ids):
    nrel = len(rows_list)

    def body(*refs):
        tables = refs[0:3]
        idx = refs[3:3 + 3 * nrel]
        outs = refs[3 + 3 * nrel:3 + 4 * nrel]
        sblk, dblk, buf0, buf1, zbuf, acc, sem0, sem1 = refs[-8:]
        bufs = (buf0, buf1)
        sems = (sem0, sem1)
        c = lax.axis_index("c")
        s = lax.axis_index("s")

        def zrow(i, _):
            for k in range(8):
                zbuf[i, pl.ds(k * 16, 16)] = jnp.zeros((16,), F32)
            return 0

        lax.fori_loop(0, ZROWS, zrow, 0)

        for r, rows in enumerate(rows_list):
            tbl = tables[table_ids[r]]
            s0, s1, d = idx[3 * r:3 * r + 3]
            out = outs[r]

            for z in range(NSTRIPE // ZROWS):
                pltpu.sync_copy(
                    zbuf, acc.at[pl.ds(s * NSTRIPE + z * ZROWS, ZROWS)])
            plsc.subcore_barrier()

            def block(b, _):
                base = s * rows + b * IBLK

                @pl.when(c == 0)
                def _():
                    pltpu.sync_copy(s0.at[pl.ds(base, IBLK)], sblk)

                @pl.when(c == 1)
                def _():
                    pltpu.sync_copy(s1.at[pl.ds(base, IBLK)], sblk)

                pltpu.sync_copy(d.at[pl.ds(base, IBLK)], dblk)
                hs = [pltpu.async_copy(tbl.at[sblk.at[0]], bufs[0], sems[0])]
                for j in range(IBLK):
                    hs[j].wait()
                    if j + 1 < IBLK:
                        hs.append(pltpu.async_copy(
                            tbl.at[sblk.at[j + 1]],
                            bufs[(j + 1) % 2], sems[(j + 1) % 2]))
                    pltpu.sync_copy(bufs[j % 2], acc.at[dblk.at[j]], add=True)
                return 0

            lax.fori_loop(0, rows // IBLK, block, 0)
            plsc.subcore_barrier()

            pltpu.sync_copy(acc.at[pl.ds(s * NSTRIPE, NSTRIPE)],
                            out.at[pl.ds(c * NT + s * NSTRIPE, NSTRIPE)])

    mesh = plsc.VectorSubcoreMesh(core_axis_name="c", subcore_axis_name="s")
    return pl.kernel(
        body,
        out_type=[jax.ShapeDtypeStruct((2 * NT, HID), F32)
                  for _ in rows_list],
        mesh=mesh,
        scratch_types=[
            pltpu.VMEM((IBLK, CHUNK), I32),
            pltpu.VMEM((IBLK, CHUNK), I32),
            pltpu.VMEM((CHUNK, HID), F32),
            pltpu.VMEM((CHUNK, HID), F32),
            pltpu.VMEM((ZROWS, HID), F32),
            pltpu.VMEM_SHARED((NT, HID), F32),
            pltpu.SemaphoreType.DMA,
            pltpu.SemaphoreType.DMA,
        ],
    )


def _pad_rows(x, n):
    return jnp.concatenate(
        [x, jnp.zeros((n - x.shape[0], x.shape[1]), x.dtype)], axis=0)


def _prep_edges(ei):
    e = ei.shape[1]
    ep = _ceil_to(e, 16 * IBLK * CHUNK)
    npad = ep - e
    src = ei[0].astype(I32)
    dst = ei[1].astype(I32)
    srcp = jnp.concatenate([src, jnp.full((npad,), N, I32)])
    dstp = jnp.concatenate(
        [dst, N + (jnp.arange(npad, dtype=I32) % (NT - N))])
    r = ep // CHUNK
    return (srcp.reshape(r, CHUNK), (srcp + NT).reshape(r, CHUNK),
            dstp.reshape(r, CHUNK), r // 16)


def kernel(params, x_cons, x_vals, x_obj, ei_cons_to_vals, ei_vals_to_cons,
           ei_vals_to_obj, ei_obj_to_vals, ei_cons_to_obj, ei_obj_to_cons):
    rels = [
        ("cv", 0, ei_cons_to_vals),
        ("ov", 2, ei_obj_to_vals),
        ("vc", 1, ei_vals_to_cons),
        ("oc", 2, ei_obj_to_cons),
        ("vo", 1, ei_vals_to_obj),
        ("co", 0, ei_cons_to_obj),
    ]
    idx_arrays = []
    rows_list = []
    for _, _, ei in rels:
        *arrs, rows = _prep_edges(ei)
        idx_arrays += arrs
        rows_list.append(rows)
    table_ids = [t for _, t, _ in rels]
    agg = _make_agg(tuple(rows_list), tuple(table_ids))

    x = {
        "cons": _encode(_pad_rows(x_cons, NT), params["enc_cons"]),
        "vals": _encode(_pad_rows(x_vals, NT), params["enc_vals"]),
        "obj": _encode(_pad_rows(x_obj, NT), params["enc_obj"]),
    }

    def layer(carry, _):
        xs = carry
        tabs = [_prep_tables(*xs[t]).reshape(2 * NT, HID)
                for t in ("cons", "vals", "obj")]
        accs = agg(*tabs, *idx_arrays)
        accs = [a.reshape(2, NT, HID) for a in accs]
        h2v, xv, gv = _combine(accs[0], accs[1], xs["vals"][0],
                               params["conv_cv"], params["conv_ov"], "batch")
        h2c, xc, gc = _combine(accs[2], accs[3], xs["cons"][0],
                               params["conv_vc"], params["conv_oc"], "batch")
        h2o, xo, go = _combine(accs[4], accs[5], xs["obj"][0],
                               params["conv_vo"], params["conv_co"], "layer")
        new = {"cons": (xc, gc), "vals": (xv, gv), "obj": (xo, go)}
        return new, (h2c, h2v)

    _, (cons, vals) = lax.scan(layer, x, None, length=NL)
    vals = _pred_head(vals, params["pred_vals_1"], params["pred_vals_2"], 2)
    cons = _pred_head(cons, params["pred_cons_1"], params["pred_cons_2"], 1)
    vals_out = jnp.transpose(vals[:, :N, :], (1, 0, 2))
    cons_out = jnp.squeeze(cons[:, :N, :], axis=-1).T
    return vals_out, cons_out
